# baseline (speedup 1.0000x reference)
import functools

import jax
import jax.numpy as jnp
from jax import lax
from jax.experimental import pallas as pl
from jax.experimental.pallas import tpu as pltpu
from jax.experimental.pallas import tpu_sc as plsc

NC, NS, LANES = 2, 16, 16
NW = NC * NS
C = 128
G = C // LANES

_DN = lax.GatherDimensionNumbers(
    offset_dims=(), collapsed_slice_dims=(0,), start_index_map=(0,))


def _lane(v, l):
    idx = jnp.full((LANES, 1), l, dtype=jnp.int32)
    return lax.gather(v, idx, _DN, (1,),
                      mode=lax.GatherScatterMode.PROMISE_IN_BOUNDS)


def _mesh():
    return plsc.VectorSubcoreMesh(
        core_axis_name="c", subcore_axis_name="s",
        num_cores=NC, num_subcores=NS)


NP = 4


@functools.lru_cache(maxsize=None)
def _make_repack(d0, d1, d2, lat):
    planes = d0 * d1
    per_w = -(-planes // (NW * NP))
    zp = d2 - 1

    @functools.partial(
        pl.kernel,
        out_type=jax.ShapeDtypeStruct((planes * zp + 4, 2 * lat),
                                      jnp.float32),
        mesh=_mesh(),
        compiler_params=pltpu.CompilerParams(use_tc_tiling_on_sc=False,
                                             needs_layout_passes=False),
        scratch_types=[
            pltpu.VMEM((NP * lat, d2), jnp.float32),
            pltpu.VMEM((NP * lat, d2), jnp.float32),
            pltpu.VMEM((NP * zp + 4, 2 * lat), jnp.float32),
            pltpu.VMEM((NP * zp + 4, 2 * lat), jnp.float32),
            pltpu.VMEM((NP, 128), jnp.int32),
            pltpu.VMEM((NP, 128), jnp.int32),
            pltpu.SemaphoreType.DMA,
            pltpu.SemaphoreType.DMA,
            pltpu.SemaphoreType.DMA,
            pltpu.SemaphoreType.DMA,
        ],
    )
    def repack(tz, tp, b0, b1, o0, o1, x0, x1, is0, is1, os0, os1):
        wid = lax.axis_index("s") * NC + lax.axis_index("c")
        blks = (b0, b1)
        obs = (o0, o1)
        oixs = (x0, x1)
        isems = (is0, is1)
        osems = (os0, os1)
        lanes_i = jnp.arange(LANES, dtype=jnp.int32)

        def istart(i, b):
            p = jnp.minimum((wid * per_w + i) * NP, planes - NP)
            pltpu.async_copy(tz.at[pl.ds(p * lat, NP * lat)], blks[b],
                             isems[b])

        def iwait(b):
            pltpu.make_async_copy(tz.at[pl.ds(0, NP * lat)], blks[b],
                                  isems[b]).wait()

        def owait(b):
            for k in range(NP):
                pltpu.make_async_copy(obs[b].at[pl.ds(k * 128, 128)],
                                      tp.at[oixs[b].at[k]],
                                      osems[b]).wait()

        def work(i, b, drain_out):
            blk, ob = blks[b], obs[b]
            iwait(b)
            if drain_out:
                owait(b)

            def pbody(pp, carry):
                rows = pp * lat + lanes_i
                obase = pp * zp
                for z in range(d2):
                    v = plsc.load_gather(blk, [rows, jnp.full(
                        (LANES,), z, dtype=jnp.int32)])
                    if z < zp:
                        ob[obase + z, pl.ds(0, lat)] = v
                    if z > 0:
                        ob[obase + z - 1, pl.ds(lat, lat)] = v
                return carry

            lax.fori_loop(0, NP, pbody, 0)
            p = jnp.minimum((wid * per_w + i) * NP, planes - NP)
            oix = oixs[b]
            for k in range(NP):
                for g in range(8):
                    r = k * 128 + g * LANES + lanes_i
                    oix[k, pl.ds(g * LANES, LANES)] = jnp.where(
                        r < NP * zp, p * zp + r,
                        planes * zp + (r - NP * zp))
            for k in range(NP):
                pltpu.async_copy(ob.at[pl.ds(k * 128, 128)],
                                 tp.at[oix.at[k]], osems[b])

        istart(0, 0)
        istart(1, 1)
        work(0, 0, drain_out=False)
        istart(2, 0)
        work(1, 1, drain_out=False)
        istart(3, 1)

        def body(i, carry):
            b = lax.rem(i, 2)

            @pl.when(b == 0)
            def _():
                work(i, 0, drain_out=True)
                istart(i + 2, 0)

            @pl.when(b == 1)
            def _():
                work(i, 1, drain_out=True)
                istart(i + 2, 1)

            return carry

        lax.fori_loop(2, per_w, body, 0)
        work(per_w, per_w % 2, drain_out=True)
        work(per_w + 1, (per_w + 1) % 2, drain_out=True)
        owait(per_w % 2)
        owait((per_w + 1) % 2)

    return repack


@functools.lru_cache(maxsize=None)
def _make_lookup(n_pts, d0, d1, d2, lat):
    zp = d2 - 1
    s1 = d1 * zp
    s2 = zp
    n_chunks = -(-n_pts // C)
    k_per_w = -(-n_chunks // NW)
    k_per_w += k_per_w % 2
    half = 4 * LANES

    @functools.partial(
        pl.kernel,
        out_type=jax.ShapeDtypeStruct((n_pts, lat), jnp.float32),
        mesh=_mesh(),
        compiler_params=pltpu.CompilerParams(use_tc_tiling_on_sc=False,
                                             needs_layout_passes=False),
        scratch_types=[
            pltpu.VMEM((3, C), jnp.float32),
            pltpu.VMEM((3, C), jnp.float32),
            pltpu.VMEM((G // 2, 2 * half), jnp.int32),
            pltpu.VMEM((G // 2, 2 * half), jnp.int32),
            pltpu.VMEM((G * half, 2 * lat), jnp.float32),
            pltpu.VMEM((G * half, 2 * lat), jnp.float32),
            pltpu.VMEM((3, C), jnp.float32),
            pltpu.VMEM((3, C), jnp.float32),
            pltpu.VMEM((C, lat), jnp.float32),
            pltpu.VMEM((C, lat), jnp.float32),
            pltpu.SemaphoreType.DMA,
            pltpu.SemaphoreType.DMA,
            pltpu.SemaphoreType.DMA,
            pltpu.SemaphoreType.DMA,
            pltpu.SemaphoreType.DMA,
        ],
    )
    def lookup(xs, ys, zs, tp, out,
               cb0, cb1, ix0, ix1, rw0, rw1, fr0, fr1, ob0, ob1,
               csem, gsem0, gsem1, osem0, osem1):
        wid = lax.axis_index("s") * NC + lax.axis_index("c")
        cbs = (cb0, cb1)
        ixs = (ix0, ix1)
        rws = (rw0, rw1)
        frs = (fr0, fr1)
        obs = (ob0, ob1)
        gsems = (gsem0, gsem1)
        osems = (osem0, osem1)

        def cstart(j, b):
            base = jnp.minimum((j * NW + wid) * C, n_pts - C)
            pltpu.async_copy(xs.at[pl.ds(base, C)], cbs[b].at[0], csem)
            pltpu.async_copy(ys.at[pl.ds(base, C)], cbs[b].at[1], csem)
            pltpu.async_copy(zs.at[pl.ds(base, C)], cbs[b].at[2], csem)

        def fire(j, b):
            del j
            cb, ixr, rw, fr, gsem = cbs[b], ixs[b], rws[b], frs[b], gsems[b]
            for _ in range(3):
                pltpu.make_async_copy(xs.at[pl.ds(0, C)], cb.at[0],
                                      csem).wait()
            for g in range(G):
                sl = pl.ds(g * LANES, LANES)
                px = cb[0, sl] * float(d0 - 1)
                py = cb[1, sl] * float(d1 - 1)
                pz = cb[2, sl] * float(d2 - 1)
                ix = jnp.clip(px.astype(jnp.int32), 0, d0 - 2)
                iy = jnp.clip(py.astype(jnp.int32), 0, d1 - 2)
                iz = jnp.clip(pz.astype(jnp.int32), 0, d2 - 2)
                fr[0, sl] = px - ix.astype(jnp.float32)
                fr[1, sl] = py - iy.astype(jnp.float32)
                fr[2, sl] = pz - iz.astype(jnp.float32)
                bidx = ix * s1 + iy * s2 + iz
                off0 = (g % 2) * half
                for cc in range(4):
                    off = (cc >> 1) * s1 + (cc & 1) * s2
                    ixr[g // 2, pl.ds(off0 + cc * LANES, LANES)] = bidx + off
                if g % 2 == 1:
                    d = g // 2
                    pltpu.async_copy(tp.at[ixr.at[d]],
                                     rw.at[pl.ds(d * 2 * half, 2 * half)],
                                     gsem)

        def compute(j, b, drain_out):
            ixr, rw, fr, ob, gsem, osem = \
                ixs[b], rws[b], frs[b], obs[b], gsems[b], osems[b]
            for _ in range(G // 2):
                pltpu.make_async_copy(tp.at[ixr.at[0]],
                                      rw.at[pl.ds(0, 2 * half)], gsem).wait()
            if drain_out:
                pltpu.make_async_copy(ob, out.at[pl.ds(0, C)], osem).wait()
            for g in range(G):
                sl = pl.ds(g * LANES, LANES)
                fxv = fr[0, sl]
                fyv = fr[1, sl]
                fzv = fr[2, sl]
                rbase = (g // 2) * 2 * half + (g % 2) * half
                for l in range(LANES):
                    r = [rw[rbase + cc * LANES + l, pl.ds(c2 * lat, lat)]
                         for cc in range(4) for c2 in range(2)]
                    bz = _lane(fzv, l)
                    s00 = r[0] + bz * (r[1] - r[0])
                    s01 = r[2] + bz * (r[3] - r[2])
                    s10 = r[4] + bz * (r[5] - r[4])
                    s11 = r[6] + bz * (r[7] - r[6])
                    by = _lane(fyv, l)
                    t0 = s00 + by * (s01 - s00)
                    t1 = s10 + by * (s11 - s10)
                    bx = _lane(fxv, l)
                    ob[g * LANES + l, :] = t0 + bx * (t1 - t0)
            base = jnp.minimum((j * NW + wid) * C, n_pts - C)
            pltpu.async_copy(ob, out.at[pl.ds(base, C)], osem)

        def owait(b):
            pltpu.make_async_copy(obs[b], out.at[pl.ds(0, C)], osems[b]).wait()

        cstart(0, 0)
        fire(0, 0)
        cstart(1, 1)
        fire(1, 1)
        cstart(2, 0)
        compute(0, 0, drain_out=False)
        fire(2, 0)
        cstart(3, 1)
        compute(1, 1, drain_out=False)

        def body(jj, carry):
            j = 2 * jj
            fire(j + 1, 1)
            cstart(j + 2, 0)
            compute(j, 0, drain_out=True)
            fire(j + 2, 0)
            cstart(j + 3, 1)
            compute(j + 1, 1, drain_out=True)
            return carry

        lax.fori_loop(1, k_per_w // 2 - 1, body, 0)

        j = k_per_w - 2
        fire(j + 1, 1)
        compute(j, 0, drain_out=True)
        compute(j + 1, 1, drain_out=True)
        owait(0)
        owait(1)

    return lookup


def kernel(unList, table):
    n = unList.shape[0]
    d0, d1, d2 = table.shape[:-1]
    lat = table.shape[-1]
    xs = unList[:, 0]
    ys = unList[:, 1]
    zs = unList[:, 2]
    tz = table.transpose(0, 1, 3, 2).reshape(d0 * d1 * lat, d2)
    tp = _make_repack(d0, d1, d2, lat)(tz)
    return _make_lookup(n, d0, d1, d2, lat)(xs, ys, zs, tp)


# speedup vs baseline: 1.0067x; 1.0067x over previous
import functools

import jax
import jax.numpy as jnp
from jax import lax
from jax.experimental import pallas as pl
from jax.experimental.pallas import tpu as pltpu
from jax.experimental.pallas import tpu_sc as plsc

NC, NS, LANES = 2, 16, 16
NW = NC * NS
C = 128
G = C // LANES

_DN = lax.GatherDimensionNumbers(
    offset_dims=(), collapsed_slice_dims=(0,), start_index_map=(0,))


def _lane(v, l):
    idx = jnp.full((LANES, 1), l, dtype=jnp.int32)
    return lax.gather(v, idx, _DN, (1,),
                      mode=lax.GatherScatterMode.PROMISE_IN_BOUNDS)


def _mesh():
    return plsc.VectorSubcoreMesh(
        core_axis_name="c", subcore_axis_name="s",
        num_cores=NC, num_subcores=NS)


@functools.lru_cache(maxsize=None)
def _make_repack(d0, d1, d2, lat):
    planes = d0 * d1
    per_w = -(-planes // NW)
    zp = d2 - 1

    @functools.partial(
        pl.kernel,
        out_type=jax.ShapeDtypeStruct((planes * zp, 2 * lat), jnp.float32),
        mesh=_mesh(),
        compiler_params=pltpu.CompilerParams(use_tc_tiling_on_sc=False,
                                             needs_layout_passes=False),
        scratch_types=[
            pltpu.VMEM((lat, d2), jnp.float32),
            pltpu.VMEM((lat, d2), jnp.float32),
            pltpu.VMEM((zp, 2 * lat), jnp.float32),
            pltpu.VMEM((zp, 2 * lat), jnp.float32),
            pltpu.SemaphoreType.DMA,
            pltpu.SemaphoreType.DMA,
            pltpu.SemaphoreType.DMA,
            pltpu.SemaphoreType.DMA,
        ],
    )
    def repack(tz, tp, b0, b1, o0, o1, is0, is1, os0, os1):
        wid = lax.axis_index("s") * NC + lax.axis_index("c")
        blks = (b0, b1)
        obs = (o0, o1)
        isems = (is0, is1)
        osems = (os0, os1)
        lanes_i = jnp.arange(LANES, dtype=jnp.int32)

        def istart(i, b):
            p = jnp.minimum(wid * per_w + i, planes - 1)
            pltpu.async_copy(tz.at[pl.ds(p * lat, lat)], blks[b], isems[b])

        def iwait(b):
            pltpu.make_async_copy(tz.at[pl.ds(0, lat)], blks[b],
                                  isems[b]).wait()

        def owait(b):
            pltpu.make_async_copy(obs[b], tp.at[pl.ds(0, zp)],
                                  osems[b]).wait()

        def work(i, b, drain_out):
            blk, ob = blks[b], obs[b]
            iwait(b)
            if drain_out:
                owait(b)

            def zbody(z, carry):
                v = plsc.load_gather(blk, [lanes_i, jnp.full(
                    (LANES,), z, dtype=jnp.int32)])
                ob[z, pl.ds(0, lat)] = v
                ob[z - 1, pl.ds(lat, lat)] = v
                return carry

            v0 = plsc.load_gather(blk, [lanes_i,
                                        jnp.zeros((LANES,), jnp.int32)])
            ob[0, pl.ds(0, lat)] = v0
            lax.fori_loop(1, zp, zbody, 0)
            vl = plsc.load_gather(blk, [lanes_i, jnp.full(
                (LANES,), zp, dtype=jnp.int32)])
            ob[zp - 1, pl.ds(lat, lat)] = vl
            p = jnp.minimum(wid * per_w + i, planes - 1)
            pltpu.async_copy(ob, tp.at[pl.ds(p * zp, zp)], osems[b])

        istart(0, 0)
        istart(1, 1)
        work(0, 0, drain_out=False)
        istart(2, 0)
        work(1, 1, drain_out=False)
        istart(3, 1)

        def body(i, carry):
            b = lax.rem(i, 2)

            @pl.when(b == 0)
            def _():
                work(i, 0, drain_out=True)
                istart(i + 2, 0)

            @pl.when(b == 1)
            def _():
                work(i, 1, drain_out=True)
                istart(i + 2, 1)

            return carry

        lax.fori_loop(2, per_w, body, 0)
        work(per_w, per_w % 2, drain_out=True)
        work(per_w + 1, (per_w + 1) % 2, drain_out=True)
        owait(per_w % 2)
        owait((per_w + 1) % 2)

    return repack


@functools.lru_cache(maxsize=None)
def _make_lookup(n_pts, d0, d1, d2, lat):
    zp = d2 - 1
    s1 = d1 * zp
    s2 = zp
    n_chunks = -(-n_pts // C)
    k_per_w = -(-n_chunks // NW)
    k_per_w += k_per_w % 2
    half = 4 * LANES

    @functools.partial(
        pl.kernel,
        out_type=jax.ShapeDtypeStruct((n_pts, lat), jnp.float32),
        mesh=_mesh(),
        compiler_params=pltpu.CompilerParams(use_tc_tiling_on_sc=False,
                                             needs_layout_passes=False),
        scratch_types=[
            pltpu.VMEM((3, C), jnp.float32),
            pltpu.VMEM((3, C), jnp.float32),
            pltpu.VMEM((G // 2, 2 * half), jnp.int32),
            pltpu.VMEM((G // 2, 2 * half), jnp.int32),
            pltpu.VMEM((G * half, 2 * lat), jnp.float32),
            pltpu.VMEM((G * half, 2 * lat), jnp.float32),
            pltpu.VMEM((3, C), jnp.float32),
            pltpu.VMEM((3, C), jnp.float32),
            pltpu.VMEM((C, lat), jnp.float32),
            pltpu.VMEM((C, lat), jnp.float32),
            pltpu.SemaphoreType.DMA,
            pltpu.SemaphoreType.DMA,
            pltpu.SemaphoreType.DMA,
            pltpu.SemaphoreType.DMA,
            pltpu.SemaphoreType.DMA,
        ],
    )
    def lookup(xs, ys, zs, tp, out,
               cb0, cb1, ix0, ix1, rw0, rw1, fr0, fr1, ob0, ob1,
               csem, gsem0, gsem1, osem0, osem1):
        wid = lax.axis_index("s") * NC + lax.axis_index("c")
        cbs = (cb0, cb1)
        ixs = (ix0, ix1)
        rws = (rw0, rw1)
        frs = (fr0, fr1)
        obs = (ob0, ob1)
        gsems = (gsem0, gsem1)
        osems = (osem0, osem1)

        def cstart(j, b):
            base = jnp.minimum((j * NW + wid) * C, n_pts - C)
            pltpu.async_copy(xs.at[pl.ds(base, C)], cbs[b].at[0], csem)
            pltpu.async_copy(ys.at[pl.ds(base, C)], cbs[b].at[1], csem)
            pltpu.async_copy(zs.at[pl.ds(base, C)], cbs[b].at[2], csem)

        def fire(j, b):
            del j
            cb, ixr, rw, fr, gsem = cbs[b], ixs[b], rws[b], frs[b], gsems[b]
            for _ in range(3):
                pltpu.make_async_copy(xs.at[pl.ds(0, C)], cb.at[0],
                                      csem).wait()
            for g in range(G):
                sl = pl.ds(g * LANES, LANES)
                px = cb[0, sl] * float(d0 - 1)
                py = cb[1, sl] * float(d1 - 1)
                pz = cb[2, sl] * float(d2 - 1)
                ix = jnp.clip(px.astype(jnp.int32), 0, d0 - 2)
                iy = jnp.clip(py.astype(jnp.int32), 0, d1 - 2)
                iz = jnp.clip(pz.astype(jnp.int32), 0, d2 - 2)
                fr[0, sl] = px - ix.astype(jnp.float32)
                fr[1, sl] = py - iy.astype(jnp.float32)
                fr[2, sl] = pz - iz.astype(jnp.float32)
                bidx = ix * s1 + iy * s2 + iz
                off0 = (g % 2) * half
                for cc in range(4):
                    off = (cc >> 1) * s1 + (cc & 1) * s2
                    ixr[g // 2, pl.ds(off0 + cc * LANES, LANES)] = bidx + off
                if g % 2 == 1:
                    d = g // 2
                    pltpu.async_copy(tp.at[ixr.at[d]],
                                     rw.at[pl.ds(d * 2 * half, 2 * half)],
                                     gsem)

        def compute(j, b, drain_out):
            ixr, rw, fr, ob, gsem, osem = \
                ixs[b], rws[b], frs[b], obs[b], gsems[b], osems[b]
            for _ in range(G // 2):
                pltpu.make_async_copy(tp.at[ixr.at[0]],
                                      rw.at[pl.ds(0, 2 * half)], gsem).wait()
            if drain_out:
                pltpu.make_async_copy(ob, out.at[pl.ds(0, C)], osem).wait()
            for g in range(G):
                sl = pl.ds(g * LANES, LANES)
                fxv = fr[0, sl]
                fyv = fr[1, sl]
                fzv = fr[2, sl]
                rbase = (g // 2) * 2 * half + (g % 2) * half
                for l in range(LANES):
                    r = [rw[rbase + cc * LANES + l, pl.ds(c2 * lat, lat)]
                         for cc in range(4) for c2 in range(2)]
                    bz = _lane(fzv, l)
                    s00 = r[0] + bz * (r[1] - r[0])
                    s01 = r[2] + bz * (r[3] - r[2])
                    s10 = r[4] + bz * (r[5] - r[4])
                    s11 = r[6] + bz * (r[7] - r[6])
                    by = _lane(fyv, l)
                    t0 = s00 + by * (s01 - s00)
                    t1 = s10 + by * (s11 - s10)
                    bx = _lane(fxv, l)
                    ob[g * LANES + l, :] = t0 + bx * (t1 - t0)
            base = jnp.minimum((j * NW + wid) * C, n_pts - C)
            pltpu.async_copy(ob, out.at[pl.ds(base, C)], osem)

        def owait(b):
            pltpu.make_async_copy(obs[b], out.at[pl.ds(0, C)], osems[b]).wait()

        cstart(0, 0)
        fire(0, 0)
        cstart(1, 1)
        fire(1, 1)
        cstart(2, 0)
        compute(0, 0, drain_out=False)
        fire(2, 0)
        cstart(3, 1)
        compute(1, 1, drain_out=False)

        def body(jj, carry):
            j = 2 * jj
            fire(j + 1, 1)
            cstart(j + 2, 0)
            compute(j, 0, drain_out=True)
            fire(j + 2, 0)
            cstart(j + 3, 1)
            compute(j + 1, 1, drain_out=True)
            return carry

        lax.fori_loop(1, k_per_w // 2 - 1, body, 0)

        j = k_per_w - 2
        fire(j + 1, 1)
        compute(j, 0, drain_out=True)
        compute(j + 1, 1, drain_out=True)
        owait(0)
        owait(1)

    return lookup


def kernel(unList, table):
    n = unList.shape[0]
    d0, d1, d2 = table.shape[:-1]
    lat = table.shape[-1]
    xs = unList[:, 0]
    ys = unList[:, 1]
    zs = unList[:, 2]
    tz = table.transpose(0, 1, 3, 2).reshape(d0 * d1 * lat, d2)
    tp = _make_repack(d0, d1, d2, lat)(tz)
    return _make_lookup(n, d0, d1, d2, lat)(xs, ys, zs, tp)
